# Initial kernel scaffold; baseline (speedup 1.0000x reference)
import jax
import jax.numpy as jnp
from jax.experimental import pallas as pl


def kernel(weight):
    raise NotImplementedError("write your pallas kernel here")


# speedup vs baseline: 1.1226x; 1.1226x over previous
import functools

import jax
import jax.numpy as jnp
from jax import lax
from jax.experimental import pallas as pl
from jax.experimental.pallas import tpu as pltpu
from jax.experimental.pallas import tpu_sc as plsc

MAX_SPAN = 255
QUERY_LENGTH = 254
KEY_LENGTH = 256
DEPTH = 512
TABLE_ROWS = MAX_SPAN * 2 - 1

_NUM_CORES = 2
_NUM_SUBCORES = 16
_NUM_WORKERS = _NUM_CORES * _NUM_SUBCORES
_Q_PER_WORKER = -(-QUERY_LENGTH // _NUM_WORKERS)


def _make_sc_kernel():
    mesh = plsc.VectorSubcoreMesh(core_axis_name="c", subcore_axis_name="s")

    @functools.partial(
        pl.kernel,
        mesh=mesh,
        out_type=jax.ShapeDtypeStruct(
            (QUERY_LENGTH, KEY_LENGTH, DEPTH), jnp.float32
        ),
        scratch_types=[
            pltpu.VMEM_SHARED((TABLE_ROWS, DEPTH), jnp.float32),
            pltpu.SemaphoreType.DMA,
        ],
        compiler_params=pltpu.CompilerParams(use_tc_tiling_on_sc=False),
    )
    def sc_kernel(w_hbm, out_hbm, shared, sem):
        cid = lax.axis_index("c")
        sid = lax.axis_index("s")
        wid = sid * _NUM_CORES + cid

        @pl.when(sid == 0)
        def _load():
            pltpu.sync_copy(w_hbm, shared)

        plsc.subcore_barrier()

        for t in range(_Q_PER_WORKER):
            q = wid + _NUM_WORKERS * t

            @pl.when(q < QUERY_LENGTH)
            def _copy():
                s = (MAX_SPAN - 2) - q
                pltpu.sync_copy(
                    shared.at[pl.ds(s, KEY_LENGTH), :], out_hbm.at[q]
                )

    return sc_kernel


def kernel(weight):
    return _make_sc_kernel()(weight)
